# baseline (speedup 1.0000x reference)
import jax
import jax.numpy as jnp
from jax.experimental import pallas as pl


_CHUNK = 512


def _split_hi_lo(v):
    hi = v.astype(jnp.bfloat16)
    lo = (v - hi.astype(jnp.float32)).astype(jnp.bfloat16)
    return hi, lo


def _chamfer_kernel(a_ref, bm_ref, out_x_ref, out_y_ref):
    n = bm_ref.shape[1]
    n_chunks = n // _CHUNK

    a_hi, a_lo = _split_hi_lo(a_ref[...])
    aa = jnp.concatenate([a_hi, a_hi, a_lo, a_lo], axis=1)
    b_hi, b_lo = _split_hi_lo(bm_ref[...])
    bb = jnp.concatenate([b_hi, b_lo, b_hi, b_lo], axis=0)

    ymin = jnp.full((n,), jnp.inf, dtype=jnp.float32)
    for i in range(n_chunks):
        a_chunk = aa[i * _CHUNK : (i + 1) * _CHUNK, :]
        t = jax.lax.dot_general(
            a_chunk,
            bb,
            (((1,), (0,)), ((), ())),
            preferred_element_type=jnp.float32,
        )
        out_x_ref[0, pl.ds(i * _CHUNK, _CHUNK)] = jnp.min(t, axis=1)
        ymin = jnp.minimum(ymin, jnp.min(t, axis=0))
    out_y_ref[0, :] = ymin


def kernel(x, y):
    b, d, n = x.shape
    f32 = jnp.float32
    k = d + 2

    xt = jnp.transpose(x, (0, 2, 1))
    nx = jnp.sum(xt * xt, axis=2, keepdims=True)
    a_full = jnp.concatenate(
        [-2.0 * xt, nx, jnp.ones((b, n, 1), f32)], axis=2
    )

    ny = jnp.sum(y * y, axis=1, keepdims=True)
    bm_full = jnp.concatenate(
        [y, jnp.ones((b, 1, n), f32), ny], axis=1
    )

    out_x, out_y = pl.pallas_call(
        _chamfer_kernel,
        grid=(b,),
        in_specs=[
            pl.BlockSpec((None, n, k), lambda i: (i, 0, 0)),
            pl.BlockSpec((None, k, n), lambda i: (i, 0, 0)),
        ],
        out_specs=[
            pl.BlockSpec((None, 1, n), lambda i: (i, 0, 0)),
            pl.BlockSpec((None, 1, n), lambda i: (i, 0, 0)),
        ],
        out_shape=[
            jax.ShapeDtypeStruct((b, 1, n), f32),
            jax.ShapeDtypeStruct((b, 1, n), f32),
        ],
    )(a_full, bm_full)

    return jnp.mean(out_x) + jnp.mean(out_y)
